# baseline (speedup 1.0000x reference)
import functools

import jax
import jax.numpy as jnp
from jax.experimental import pallas as pl
from jax.experimental.pallas import tpu as pltpu


def _mb_kernel(batch, r, grid, u_ref, out_any, bank_any, bank_ref,
               outbank_ref, newbank_any, sem_head, sem_tail):
    i = pl.program_id(0)

    @pl.when(i == 0)
    def _():
        upd = u_ref[0] != 0

        def head_from_output():
            pltpu.make_async_copy(
                out_any, newbank_any.at[pl.ds(0, batch)], sem_head).start()

        def head_from_bank():
            pltpu.make_async_copy(
                bank_any.at[pl.ds(0, batch)],
                newbank_any.at[pl.ds(0, batch)], sem_head).start()

        jax.lax.cond(upd, head_from_output, head_from_bank)
        pltpu.make_async_copy(
            bank_any.at[pl.ds(batch, bank_any.shape[0] - batch)],
            newbank_any.at[pl.ds(batch, bank_any.shape[0] - batch)],
            sem_tail).start()

    outbank_ref[...] = bank_ref[...].T

    @pl.when(i == grid - 1)
    def _():
        pltpu.make_async_copy(
            out_any, newbank_any.at[pl.ds(0, batch)], sem_head).wait()
        pltpu.make_async_copy(
            bank_any.at[pl.ds(batch, bank_any.shape[0] - batch)],
            newbank_any.at[pl.ds(batch, bank_any.shape[0] - batch)],
            sem_tail).wait()


def kernel(output, bank, update):
    size, dim = bank.shape
    batch = output.shape[0]
    r = 16384
    grid = size // r
    u = jnp.asarray(update, jnp.int32).reshape(1)

    body = functools.partial(_mb_kernel, batch, r, grid)
    out_bank, new_bank = pl.pallas_call(
        body,
        grid=(grid,),
        in_specs=[
            pl.BlockSpec(memory_space=pltpu.SMEM),
            pl.BlockSpec(memory_space=pl.ANY),
            pl.BlockSpec(memory_space=pl.ANY),
            pl.BlockSpec((r, dim), lambda i: (i, 0)),
        ],
        out_specs=[
            pl.BlockSpec((dim, r), lambda i: (0, i)),
            pl.BlockSpec(memory_space=pl.ANY),
        ],
        out_shape=[
            jax.ShapeDtypeStruct((dim, size), bank.dtype),
            jax.ShapeDtypeStruct((size, dim), bank.dtype),
        ],
        scratch_shapes=[
            pltpu.SemaphoreType.DMA,
            pltpu.SemaphoreType.DMA,
        ],
    )(u, output, bank, bank)
    return (output, out_bank, new_bank)


# speedup vs baseline: 29.9359x; 29.9359x over previous
import functools

import jax
import jax.numpy as jnp
from jax.experimental import pallas as pl
from jax.experimental.pallas import tpu as pltpu


def _mb_kernel(batch, r, u_ref, out_in_ref, bank_ref,
               out_copy_ref, outbank_ref, newbank_ref):
    i = pl.program_id(0)
    blk = bank_ref[...]
    outbank_ref[...] = blk.T
    upd = u_ref[0] != 0

    @pl.when(i == 0)
    def _():
        out_full = out_in_ref[...]
        out_copy_ref[...] = out_full
        head = jnp.where(upd, out_full, blk[:batch])
        if r > batch:
            newbank_ref[...] = jnp.concatenate([head, blk[batch:]], axis=0)
        else:
            newbank_ref[...] = head

    @pl.when(i != 0)
    def _():
        newbank_ref[...] = blk


def kernel(output, bank, update):
    size, dim = bank.shape
    batch = output.shape[0]
    r = 16384
    grid = size // r
    u = jnp.asarray(update, jnp.int32).reshape(1)

    body = functools.partial(_mb_kernel, batch, r)
    out_copy, out_bank, new_bank = pl.pallas_call(
        body,
        grid=(grid,),
        in_specs=[
            pl.BlockSpec(memory_space=pltpu.SMEM),
            pl.BlockSpec((batch, dim), lambda i: (0, 0)),
            pl.BlockSpec((r, dim), lambda i: (i, 0)),
        ],
        out_specs=[
            pl.BlockSpec((batch, dim), lambda i: (0, 0)),
            pl.BlockSpec((dim, r), lambda i: (0, i)),
            pl.BlockSpec((r, dim), lambda i: (i, 0)),
        ],
        out_shape=[
            jax.ShapeDtypeStruct((batch, dim), output.dtype),
            jax.ShapeDtypeStruct((dim, size), bank.dtype),
            jax.ShapeDtypeStruct((size, dim), bank.dtype),
        ],
        compiler_params=pltpu.CompilerParams(
            vmem_limit_bytes=100 * 1024 * 1024),
    )(u, output, bank)
    return (out_copy, out_bank, new_bank)
